# baseline (speedup 1.0000x reference)
import functools
import math

import jax
import jax.numpy as jnp
from jax import lax
from jax.experimental import pallas as pl
from jax.experimental.pallas import tpu as pltpu
from jax.experimental.pallas import tpu_sc as plsc

D_MODEL = 64
SCALE = math.sqrt(D_MODEL)
PAD_W = 128
LANES = 16

NUM_CORES = 2
NUM_SUBCORES = 16
NUM_WORKERS = NUM_CORES * NUM_SUBCORES

PREP_B = 8192
CHUNK = 256


def _prep_table(table_t, vocab):
    grid = (vocab + PREP_B - 1) // PREP_B

    def body(t_ref, o_ref):
        o_ref[:, 0:D_MODEL] = t_ref[...].T * SCALE

    return pl.pallas_call(
        body,
        grid=(grid,),
        in_specs=[
            pl.BlockSpec((D_MODEL, PREP_B), lambda i: (0, i)),
        ],
        out_specs=pl.BlockSpec((PREP_B, PAD_W), lambda i: (i, 0)),
        out_shape=jax.ShapeDtypeStruct((vocab, PAD_W), jnp.float32),
        compiler_params=pltpu.CompilerParams(
            dimension_semantics=("parallel",)
        ),
    )(table_t)


def _gather_t(t8, idx2d, batch, nchunks):
    seq = idx2d.size // batch
    chunks_per_s = batch // CHUNK
    mesh = plsc.VectorSubcoreMesh(core_axis_name="c", subcore_axis_name="s")

    @functools.partial(
        pl.kernel,
        out_type=jax.ShapeDtypeStruct((seq * D_MODEL, batch), jnp.float32),
        mesh=mesh,
        compiler_params=pltpu.CompilerParams(needs_layout_passes=False),
        scratch_types=[
            pltpu.VMEM((nchunks * CHUNK,), jnp.int32),
            pltpu.VMEM((CHUNK, PAD_W), jnp.float32),
            pltpu.VMEM((CHUNK, PAD_W), jnp.float32),
            pltpu.VMEM((D_MODEL, CHUNK), jnp.float32),
            pltpu.VMEM((D_MODEL, CHUNK), jnp.float32),
            pltpu.SemaphoreType.DMA,
            pltpu.SemaphoreType.DMA,
            pltpu.SemaphoreType.DMA,
            pltpu.SemaphoreType.DMA,
        ],
    )
    def kern(t8_hbm, i_hbm, o_hbm, idx_v, g0, g1, t0, t1, sg0, sg1, so0, so1):
        wid = lax.axis_index("s") * NUM_CORES + lax.axis_index("c")

        pltpu.sync_copy(i_hbm.at[wid], idx_v)

        def start_gather(c, g, sem):
            pltpu.async_copy(t8_hbm.at[idx_v.at[pl.ds(c * CHUNK, CHUNK)]], g, sem)

        def wait_gather(c, g, sem):
            pltpu.make_async_copy(
                t8_hbm.at[idx_v.at[pl.ds(c * CHUNK, CHUNK)]], g, sem
            ).wait()

        def dst_slice(c):
            gchunk = wid * nchunks + c
            s = gchunk // chunks_per_s
            b0 = (gchunk % chunks_per_s) * CHUNK
            return o_hbm.at[pl.ds(s * D_MODEL, D_MODEL), pl.ds(b0, CHUNK)]

        def transpose_fill(g, t):
            @pl.loop(0, D_MODEL)
            def _(d):
                dvec = jnp.zeros((LANES,), jnp.int32) + d
                for i0 in range(0, CHUNK, LANES):
                    ivec = jnp.arange(LANES, dtype=jnp.int32) + i0
                    vec = plsc.load_gather(g, [ivec, dvec])
                    plsc.store_scatter(t, [dvec, ivec], vec)

        def start_out(c, t, sem):
            pltpu.async_copy(t, dst_slice(c), sem)

        def wait_out(c, t, sem):
            pltpu.make_async_copy(t, dst_slice(c), sem).wait()

        start_gather(0, g0, sg0)
        start_gather(1, g1, sg1)

        @pl.loop(0, nchunks, step=2)
        def _(c):
            wait_gather(c, g0, sg0)

            @pl.when(c >= 2)
            def _():
                wait_out(c - 2, t0, so0)

            transpose_fill(g0, t0)
            start_out(c, t0, so0)

            @pl.when(c + 2 < nchunks)
            def _():
                start_gather(c + 2, g0, sg0)

            wait_gather(c + 1, g1, sg1)

            @pl.when(c >= 2)
            def _():
                wait_out(c - 1, t1, so1)

            transpose_fill(g1, t1)
            start_out(c + 1, t1, so1)

            @pl.when(c + 3 < nchunks)
            def _():
                start_gather(c + 3, g1, sg1)

        wait_out(nchunks - 2, t0, so0)
        wait_out(nchunks - 1, t1, so1)

    return kern(t8, idx2d)


def kernel(x, table):
    b, s = x.shape
    n = b * s
    vocab, d = table.shape
    nchunks = n // (NUM_WORKERS * CHUNK)
    idx2d = jnp.transpose(x).astype(jnp.int32).reshape(NUM_WORKERS, nchunks * CHUNK)
    t8 = _prep_table(jnp.transpose(table), vocab)
    out2d = _gather_t(t8, idx2d, b, nchunks)
    return jnp.transpose(out2d.reshape(s, D_MODEL, b), (2, 0, 1))


# speedup vs baseline: 1.0324x; 1.0324x over previous
import functools
import math

import jax
import jax.numpy as jnp
from jax import lax
from jax.experimental import pallas as pl
from jax.experimental.pallas import tpu as pltpu
from jax.experimental.pallas import tpu_sc as plsc

D_MODEL = 64
SCALE = math.sqrt(D_MODEL)
PAD_W = 128

NUM_CORES = 2
NUM_SUBCORES = 16
NUM_WORKERS = NUM_CORES * NUM_SUBCORES

PREP_B = 8192
CHUNK = 320
LANES = 16


def _prep_table(table_t, vocab):
    grid = (vocab + PREP_B - 1) // PREP_B

    def body(t_ref, o_ref):
        o_ref[:, 0:D_MODEL] = t_ref[...].T * SCALE

    return pl.pallas_call(
        body,
        grid=(grid,),
        in_specs=[
            pl.BlockSpec((D_MODEL, PREP_B), lambda i: (0, i)),
        ],
        out_specs=pl.BlockSpec((PREP_B, PAD_W), lambda i: (i, 0)),
        out_shape=jax.ShapeDtypeStruct((vocab, PAD_W), jnp.float32),
        compiler_params=pltpu.CompilerParams(
            dimension_semantics=("parallel",)
        ),
    )(table_t)


def _gather64(t8, idx3d, n, nchunks):
    mesh = plsc.VectorSubcoreMesh(core_axis_name="c", subcore_axis_name="s")

    @functools.partial(
        pl.kernel,
        out_type=jax.ShapeDtypeStruct((n, PAD_W), jnp.float32),
        mesh=mesh,
        scratch_types=[
            pltpu.VMEM((nchunks * CHUNK,), jnp.int32),
            pltpu.VMEM((CHUNK, PAD_W), jnp.float32),
            pltpu.VMEM((CHUNK, PAD_W), jnp.float32),
            pltpu.SemaphoreType.DMA,
            pltpu.SemaphoreType.DMA,
        ],
    )
    def kern(t8_hbm, i_hbm, o_hbm, idx_v, g0, g1, sem0, sem1):
        wid = lax.axis_index("s") * NUM_CORES + lax.axis_index("c")
        per_w = nchunks * CHUNK
        base = wid * per_w

        pltpu.sync_copy(i_hbm.at[wid], idx_v)

        def start_gather(c, g, sem):
            pltpu.async_copy(t8_hbm.at[idx_v.at[pl.ds(c * CHUNK, CHUNK)]], g, sem)

        def wait_gather(c, g, sem):
            pltpu.make_async_copy(
                t8_hbm.at[idx_v.at[pl.ds(c * CHUNK, CHUNK)]], g, sem
            ).wait()

        def copy_out(c, g):
            pltpu.sync_copy(g, o_hbm.at[pl.ds(base + c * CHUNK, CHUNK)])

        start_gather(0, g0, sem0)
        start_gather(1, g1, sem1)

        @pl.loop(0, nchunks, step=2)
        def _(c):
            wait_gather(c, g0, sem0)
            copy_out(c, g0)

            @pl.when(c + 2 < nchunks)
            def _():
                start_gather(c + 2, g0, sem0)

            wait_gather(c + 1, g1, sem1)
            copy_out(c + 1, g1)

            @pl.when(c + 3 < nchunks)
            def _():
                start_gather(c + 3, g1, sem1)

    return kern(t8, idx3d)


K_SPLIT = 4
TR_B = 512


def _transform(g, prev, k, sk, batch, seq):
    nb = batch // TR_B

    def body(g_ref, p_ref, o_ref):
        del p_ref
        o_ref[...] = g_ref[:, 0:D_MODEL].T

    in_specs = [
        pl.BlockSpec((TR_B, PAD_W), lambda si, bi: (si * nb + bi, 0)),
        pl.BlockSpec(memory_space=pl.ANY),
    ]
    out_specs = pl.BlockSpec(
        (D_MODEL, TR_B), lambda si, bi: (k * sk + si, bi)
    )
    return pl.pallas_call(
        body,
        grid=(sk, nb),
        in_specs=in_specs,
        out_specs=out_specs,
        out_shape=jax.ShapeDtypeStruct((seq * D_MODEL, batch), jnp.float32),
        input_output_aliases={1: 0},
        compiler_params=pltpu.CompilerParams(
            dimension_semantics=("parallel", "parallel")
        ),
    )(g, prev)


def kernel(x, table):
    b, s = x.shape
    vocab, d = table.shape
    t8 = _prep_table(jnp.transpose(table), vocab)
    xt = jnp.transpose(x).astype(jnp.int32)
    sk = s // K_SPLIT
    nk = sk * b
    nchunks = nk // (NUM_WORKERS * CHUNK)
    out2d = jnp.zeros((0,), jnp.float32)
    for k in range(K_SPLIT):
        idx_k = xt[k * sk:(k + 1) * sk].reshape(NUM_WORKERS, nchunks * CHUNK)
        g_k = _gather64(t8, idx_k, nk, nchunks)
        if k == 0:
            out2d = _transform_first(g_k, sk, b, s)
        else:
            out2d = _transform(g_k, out2d, k, sk, b, s)
    return jnp.transpose(out2d.reshape(s, D_MODEL, b), (2, 0, 1))


def _transform_first(g, sk, batch, seq):
    nb = batch // TR_B

    def body(g_ref, o_ref):
        o_ref[...] = g_ref[:, 0:D_MODEL].T

    return pl.pallas_call(
        body,
        grid=(sk, nb),
        in_specs=[
            pl.BlockSpec((TR_B, PAD_W), lambda si, bi: (si * nb + bi, 0)),
        ],
        out_specs=pl.BlockSpec((D_MODEL, TR_B), lambda si, bi: (si, bi)),
        out_shape=jax.ShapeDtypeStruct((seq * D_MODEL, batch), jnp.float32),
        compiler_params=pltpu.CompilerParams(
            dimension_semantics=("parallel", "parallel")
        ),
    )(g)


# speedup vs baseline: 1.7078x; 1.6543x over previous
import functools
import math

import jax
import jax.numpy as jnp
from jax import lax
from jax.experimental import pallas as pl
from jax.experimental.pallas import tpu as pltpu
from jax.experimental.pallas import tpu_sc as plsc

D_MODEL = 64
SCALE = math.sqrt(D_MODEL)
PAD_W = 128

NUM_CORES = 2
NUM_SUBCORES = 16
NUM_WORKERS = NUM_CORES * NUM_SUBCORES

PREP_B = 8192
CHUNK = 320
LANES = 16


def _prep_table(table_t, vocab):
    grid = (vocab + PREP_B - 1) // PREP_B

    def body(t_ref, o_ref):
        o_ref[:, 0:D_MODEL] = t_ref[...].T * SCALE

    return pl.pallas_call(
        body,
        grid=(grid,),
        in_specs=[
            pl.BlockSpec((D_MODEL, PREP_B), lambda i: (0, i)),
        ],
        out_specs=pl.BlockSpec((PREP_B, PAD_W), lambda i: (i, 0)),
        out_shape=jax.ShapeDtypeStruct((vocab, PAD_W), jnp.float32),
        compiler_params=pltpu.CompilerParams(
            dimension_semantics=("parallel",)
        ),
    )(table_t)


def _gather64(t8, idx3d, n, nchunks):
    mesh = plsc.VectorSubcoreMesh(core_axis_name="c", subcore_axis_name="s")

    @functools.partial(
        pl.kernel,
        out_type=jax.ShapeDtypeStruct((n, PAD_W), jnp.float32),
        mesh=mesh,
        scratch_types=[
            pltpu.VMEM((nchunks * CHUNK,), jnp.int32),
            pltpu.VMEM((CHUNK, PAD_W), jnp.float32),
            pltpu.VMEM((CHUNK, PAD_W), jnp.float32),
            pltpu.SemaphoreType.DMA,
            pltpu.SemaphoreType.DMA,
        ],
    )
    def kern(t8_hbm, i_hbm, o_hbm, idx_v, g0, g1, sem0, sem1):
        wid = lax.axis_index("s") * NUM_CORES + lax.axis_index("c")
        per_w = nchunks * CHUNK
        base = wid * per_w

        pltpu.sync_copy(i_hbm.at[wid], idx_v)

        def start_gather(c, g, sem):
            pltpu.async_copy(t8_hbm.at[idx_v.at[pl.ds(c * CHUNK, CHUNK)]], g, sem)

        def wait_gather(c, g, sem):
            pltpu.make_async_copy(
                t8_hbm.at[idx_v.at[pl.ds(c * CHUNK, CHUNK)]], g, sem
            ).wait()

        def copy_out(c, g):
            pltpu.sync_copy(g, o_hbm.at[pl.ds(base + c * CHUNK, CHUNK)])

        start_gather(0, g0, sem0)
        start_gather(1, g1, sem1)

        @pl.loop(0, nchunks, step=2)
        def _(c):
            wait_gather(c, g0, sem0)
            copy_out(c, g0)

            @pl.when(c + 2 < nchunks)
            def _():
                start_gather(c + 2, g0, sem0)

            wait_gather(c + 1, g1, sem1)
            copy_out(c + 1, g1)

            @pl.when(c + 3 < nchunks)
            def _():
                start_gather(c + 3, g1, sem1)

    return kern(t8, idx3d)


K_SPLIT = 4
TR_B = 2048


def _transform(g, prev, k, sk, batch, seq):
    nb = batch // TR_B

    def body(g_ref, p_ref, o_ref):
        del p_ref
        o_ref[...] = g_ref[:, 0:D_MODEL].T

    in_specs = [
        pl.BlockSpec((TR_B, PAD_W), lambda si, bi: (si * nb + bi, 0)),
        pl.BlockSpec(memory_space=pl.ANY),
    ]
    out_specs = pl.BlockSpec(
        (D_MODEL, TR_B), lambda si, bi: (k * sk + si, bi)
    )
    return pl.pallas_call(
        body,
        grid=(sk, nb),
        in_specs=in_specs,
        out_specs=out_specs,
        out_shape=jax.ShapeDtypeStruct((seq * D_MODEL, batch), jnp.float32),
        input_output_aliases={1: 0},
        compiler_params=pltpu.CompilerParams(
            dimension_semantics=("parallel", "parallel")
        ),
    )(g, prev)


def kernel(x, table):
    b, s = x.shape
    vocab, d = table.shape
    t8 = _prep_table(jnp.transpose(table), vocab)
    xt = jnp.transpose(x).astype(jnp.int32)
    sk = s // K_SPLIT
    nk = sk * b
    nchunks = nk // (NUM_WORKERS * CHUNK)
    out2d = jnp.zeros((0,), jnp.float32)
    for k in range(K_SPLIT):
        idx_k = xt[k * sk:(k + 1) * sk].reshape(NUM_WORKERS, nchunks * CHUNK)
        g_k = _gather64(t8, idx_k, nk, nchunks)
        if k == 0:
            out2d = _transform_first(g_k, sk, b, s)
        else:
            out2d = _transform(g_k, out2d, k, sk, b, s)
    return jnp.transpose(out2d.reshape(s, D_MODEL, b), (2, 0, 1))


def _transform_first(g, sk, batch, seq):
    nb = batch // TR_B

    def body(g_ref, o_ref):
        o_ref[...] = g_ref[:, 0:D_MODEL].T

    return pl.pallas_call(
        body,
        grid=(sk, nb),
        in_specs=[
            pl.BlockSpec((TR_B, PAD_W), lambda si, bi: (si * nb + bi, 0)),
        ],
        out_specs=pl.BlockSpec((D_MODEL, TR_B), lambda si, bi: (si, bi)),
        out_shape=jax.ShapeDtypeStruct((seq * D_MODEL, batch), jnp.float32),
        compiler_params=pltpu.CompilerParams(
            dimension_semantics=("parallel", "parallel")
        ),
    )(g)


# speedup vs baseline: 2.0482x; 1.1993x over previous
import functools
import math

import jax
import jax.numpy as jnp
from jax import lax
from jax.experimental import pallas as pl
from jax.experimental.pallas import tpu as pltpu
from jax.experimental.pallas import tpu_sc as plsc

D_MODEL = 64
SCALE = math.sqrt(D_MODEL)
PAD_W = 128

NUM_CORES = 2
NUM_SUBCORES = 16
NUM_WORKERS = NUM_CORES * NUM_SUBCORES

PREP_B = 16384
CHUNK = 160
NBUF = 4
LANES = 16


def _prep_table(table_t, vocab):
    grid = (vocab + PREP_B - 1) // PREP_B

    def body(t_ref, o_ref):
        o_ref[:, 0:D_MODEL] = t_ref[...].T * SCALE

    return pl.pallas_call(
        body,
        grid=(grid,),
        in_specs=[
            pl.BlockSpec((D_MODEL, PREP_B), lambda i: (0, i)),
        ],
        out_specs=pl.BlockSpec((PREP_B, PAD_W), lambda i: (i, 0)),
        out_shape=jax.ShapeDtypeStruct((vocab, PAD_W), jnp.float32),
        compiler_params=pltpu.CompilerParams(
            dimension_semantics=("parallel",)
        ),
    )(table_t)


def _gather64(t8, idx3d, n, nchunks):
    mesh = plsc.VectorSubcoreMesh(core_axis_name="c", subcore_axis_name="s")

    @functools.partial(
        pl.kernel,
        out_type=jax.ShapeDtypeStruct((n, PAD_W), jnp.float32),
        mesh=mesh,
        scratch_types=(
            [pltpu.VMEM((nchunks * CHUNK,), jnp.int32)]
            + [pltpu.VMEM((CHUNK, PAD_W), jnp.float32) for _ in range(NBUF)]
            + [pltpu.SemaphoreType.DMA for _ in range(NBUF)]
        ),
    )
    def kern(t8_hbm, i_hbm, o_hbm, idx_v, *bufs_sems):
        gbufs = bufs_sems[:NBUF]
        sems = bufs_sems[NBUF:]
        wid = lax.axis_index("s") * NUM_CORES + lax.axis_index("c")
        per_w = nchunks * CHUNK
        base = wid * per_w

        pltpu.sync_copy(i_hbm.at[wid], idx_v)

        def start_gather(c, g, sem):
            pltpu.async_copy(t8_hbm.at[idx_v.at[pl.ds(c * CHUNK, CHUNK)]], g, sem)

        def wait_gather(c, g, sem):
            pltpu.make_async_copy(
                t8_hbm.at[idx_v.at[pl.ds(c * CHUNK, CHUNK)]], g, sem
            ).wait()

        def copy_out(c, g):
            pltpu.sync_copy(g, o_hbm.at[pl.ds(base + c * CHUNK, CHUNK)])

        for b in range(NBUF):
            start_gather(b, gbufs[b], sems[b])

        @pl.loop(0, nchunks, step=NBUF)
        def _(c):
            for b in range(NBUF):
                wait_gather(c + b, gbufs[b], sems[b])
                copy_out(c + b, gbufs[b])

                @pl.when(c + b + NBUF < nchunks)
                def _():
                    start_gather(c + b + NBUF, gbufs[b], sems[b])

    return kern(t8, idx3d)


def kernel(x, table):
    b, s = x.shape
    n = b * s
    vocab, d = table.shape
    nchunks = n // (NUM_WORKERS * CHUNK)
    idx3d = x.astype(jnp.int32).reshape(NUM_WORKERS, nchunks * CHUNK)
    t8 = _prep_table(jnp.transpose(table), vocab)
    out128 = _gather64(t8, idx3d, n, nchunks)
    return out128[:, 0:D_MODEL].reshape(b, s, D_MODEL)


# speedup vs baseline: 2.0513x; 1.0015x over previous
import functools
import math

import jax
import jax.numpy as jnp
from jax import lax
from jax.experimental import pallas as pl
from jax.experimental.pallas import tpu as pltpu
from jax.experimental.pallas import tpu_sc as plsc

D_MODEL = 64
SCALE = math.sqrt(D_MODEL)
PAD_W = 128

NUM_CORES = 2
NUM_SUBCORES = 16
NUM_WORKERS = NUM_CORES * NUM_SUBCORES

PREP_B = 16384
CHUNK = 128
NBUF = 5
LANES = 16


def _prep_table(table_t, vocab):
    grid = (vocab + PREP_B - 1) // PREP_B

    def body(t_ref, o_ref):
        o_ref[:, 0:D_MODEL] = t_ref[...].T * SCALE

    return pl.pallas_call(
        body,
        grid=(grid,),
        in_specs=[
            pl.BlockSpec((D_MODEL, PREP_B), lambda i: (0, i)),
        ],
        out_specs=pl.BlockSpec((PREP_B, PAD_W), lambda i: (i, 0)),
        out_shape=jax.ShapeDtypeStruct((vocab, PAD_W), jnp.float32),
        compiler_params=pltpu.CompilerParams(
            dimension_semantics=("parallel",)
        ),
    )(table_t)


def _gather64(t8, idx3d, n, nchunks):
    mesh = plsc.VectorSubcoreMesh(core_axis_name="c", subcore_axis_name="s")

    @functools.partial(
        pl.kernel,
        out_type=jax.ShapeDtypeStruct((n, PAD_W), jnp.float32),
        mesh=mesh,
        scratch_types=(
            [pltpu.VMEM((nchunks * CHUNK,), jnp.int32)]
            + [pltpu.VMEM((CHUNK, PAD_W), jnp.float32) for _ in range(NBUF)]
            + [pltpu.SemaphoreType.DMA for _ in range(NBUF)]
        ),
    )
    def kern(t8_hbm, i_hbm, o_hbm, idx_v, *bufs_sems):
        gbufs = bufs_sems[:NBUF]
        sems = bufs_sems[NBUF:]
        wid = lax.axis_index("s") * NUM_CORES + lax.axis_index("c")
        per_w = nchunks * CHUNK
        base = wid * per_w

        pltpu.sync_copy(i_hbm.at[wid], idx_v)

        def start_gather(c, g, sem):
            pltpu.async_copy(t8_hbm.at[idx_v.at[pl.ds(c * CHUNK, CHUNK)]], g, sem)

        def wait_gather(c, g, sem):
            pltpu.make_async_copy(
                t8_hbm.at[idx_v.at[pl.ds(c * CHUNK, CHUNK)]], g, sem
            ).wait()

        def copy_out(c, g):
            pltpu.sync_copy(g, o_hbm.at[pl.ds(base + c * CHUNK, CHUNK)])

        for b in range(NBUF):
            start_gather(b, gbufs[b], sems[b])

        @pl.loop(0, nchunks, step=NBUF)
        def _(c):
            for b in range(NBUF):
                wait_gather(c + b, gbufs[b], sems[b])
                copy_out(c + b, gbufs[b])

                @pl.when(c + b + NBUF < nchunks)
                def _():
                    start_gather(c + b + NBUF, gbufs[b], sems[b])

    return kern(t8, idx3d)


def kernel(x, table):
    b, s = x.shape
    n = b * s
    vocab, d = table.shape
    nchunks = n // (NUM_WORKERS * CHUNK)
    idx3d = x.astype(jnp.int32).reshape(NUM_WORKERS, nchunks * CHUNK)
    t8 = _prep_table(jnp.transpose(table), vocab)
    out128 = _gather64(t8, idx3d, n, nchunks)
    return out128[:, 0:D_MODEL].reshape(b, s, D_MODEL)
